# baseline (speedup 1.0000x reference)
import functools

import jax
import jax.numpy as jnp
from jax import lax
from jax.experimental import pallas as pl
from jax.experimental.pallas import tpu as pltpu
from jax.experimental.pallas import tpu_sc as plsc

_LANES = 16
_ROWS = 16
_SPLIT = 6144


def _tc_body(emb_ref, out_ref, *, scale):
    out_ref[...] = emb_ref[...] * scale


def _sc_scale_body(r0, n_chunks, dim, scale, emb_hbm, out_hbm, a0, a1, b0, b1,
                   si0, si1, so0, so1):
    nc = 2
    wid = lax.axis_index("s") * nc + lax.axis_index("c")
    base = r0 + wid * (n_chunks * _ROWS)
    ins, outs = [a0, a1], [b0, b1]
    sin, sout = [si0, si1], [so0, so1]

    def issue_in(c, b):
        pltpu.async_copy(emb_hbm.at[pl.ds(base + c * _ROWS, _ROWS)], ins[b], sin[b])

    def wait_in(b):
        pltpu.make_async_copy(emb_hbm.at[pl.ds(base, _ROWS)], ins[b], sin[b]).wait()

    def issue_out(c, b):
        pltpu.async_copy(outs[b], out_hbm.at[pl.ds((base - r0) + c * _ROWS, _ROWS)], sout[b])

    def wait_out(b):
        pltpu.make_async_copy(outs[b], out_hbm.at[pl.ds(0, _ROWS)], sout[b]).wait()

    def compute(b):
        src, dst = ins[b], outs[b]

        @plsc.parallel_loop(0, _ROWS)
        def _(r):
            for k in range(dim // _LANES):
                sl = pl.ds(k * _LANES, _LANES)
                dst[r, sl] = src[r, sl] * scale

    issue_in(0, 0)
    issue_in(1, 1)
    wait_in(0)
    compute(0)
    issue_out(0, 0)
    issue_in(2, 0)
    wait_in(1)
    compute(1)
    issue_out(1, 1)
    issue_in(3, 1)

    def gbody(g, _):
        c0 = 2 * g
        wait_in(0)
        wait_out(0)
        compute(0)
        issue_out(c0, 0)
        issue_in(c0 + 2, 0)
        wait_in(1)
        wait_out(1)
        compute(1)
        issue_out(c0 + 1, 1)
        issue_in(c0 + 3, 1)
        return 0

    lax.fori_loop(1, n_chunks // 2 - 1, gbody, 0)

    wait_in(0)
    wait_out(0)
    compute(0)
    issue_out(n_chunks - 2, 0)
    wait_in(1)
    wait_out(1)
    compute(1)
    issue_out(n_chunks - 1, 1)
    wait_out(0)
    wait_out(1)


def kernel(x, emb):
    seq_len = x.shape[1]
    dim = emb.shape[1]
    scale = dim ** (-0.5)
    n_workers = 32
    sc_rows = seq_len - _SPLIT
    n_chunks = sc_rows // (n_workers * _ROWS)

    blk = 2048
    tc_out = pl.pallas_call(
        functools.partial(_tc_body, scale=scale),
        grid=(_SPLIT // blk,),
        in_specs=[pl.BlockSpec((blk, dim), lambda i: (i, 0))],
        out_specs=pl.BlockSpec((blk, dim), lambda i: (i, 0)),
        out_shape=jax.ShapeDtypeStruct((_SPLIT, dim), emb.dtype),
    )(emb)

    mesh = plsc.VectorSubcoreMesh(core_axis_name="c", subcore_axis_name="s")
    sc_call = pl.kernel(
        functools.partial(_sc_scale_body, _SPLIT, n_chunks, dim, scale),
        mesh=mesh,
        out_type=jax.ShapeDtypeStruct((sc_rows, dim), emb.dtype),
        scratch_types=[
            pltpu.VMEM((_ROWS, dim), jnp.float32),
            pltpu.VMEM((_ROWS, dim), jnp.float32),
            pltpu.VMEM((_ROWS, dim), jnp.float32),
            pltpu.VMEM((_ROWS, dim), jnp.float32),
            pltpu.SemaphoreType.DMA,
            pltpu.SemaphoreType.DMA,
            pltpu.SemaphoreType.DMA,
            pltpu.SemaphoreType.DMA,
        ],
    )
    sc_out = sc_call(emb)
    return jnp.concatenate([tc_out, sc_out], axis=0)


# speedup vs baseline: 2.4362x; 2.4362x over previous
import functools

import jax
import jax.numpy as jnp
from jax.experimental import pallas as pl
from jax.experimental.pallas import tpu as pltpu

_R = 512


def _body(emb_hbm, out_hbm, a0, a1, b0, b1, si0, si1, so0, so1,
          *, n_chunks, scale):
    ins, outs = [a0, a1], [b0, b1]
    sin, sout = [si0, si1], [so0, so1]
    h_in = [None] * n_chunks
    h_out = [None] * n_chunks

    def issue_in(c):
        b = c & 1
        h = pltpu.make_async_copy(emb_hbm.at[pl.ds(c * _R, _R)], ins[b], sin[b])
        h.start()
        return h

    def issue_out(c):
        b = c & 1
        h = pltpu.make_async_copy(outs[b], out_hbm.at[pl.ds(c * _R, _R)], sout[b])
        h.start()
        return h

    h_in[0] = issue_in(0)
    h_in[1] = issue_in(1)
    for c in range(n_chunks):
        b = c & 1
        h_in[c].wait()
        if c >= 2:
            h_out[c - 2].wait()
        outs[b][...] = ins[b][...] * scale
        h_out[c] = issue_out(c)
        if c + 2 < n_chunks:
            h_in[c + 2] = issue_in(c + 2)
    h_out[n_chunks - 2].wait()
    h_out[n_chunks - 1].wait()


def kernel(x, emb):
    seq_len = x.shape[1]
    dim = emb.shape[1]
    scale = dim ** (-0.5)
    n_chunks = seq_len // _R
    return pl.pallas_call(
        functools.partial(_body, n_chunks=n_chunks, scale=scale),
        in_specs=[pl.BlockSpec(memory_space=pltpu.MemorySpace.HBM)],
        out_specs=pl.BlockSpec(memory_space=pltpu.MemorySpace.HBM),
        out_shape=jax.ShapeDtypeStruct((seq_len, dim), emb.dtype),
        scratch_shapes=[
            pltpu.VMEM((_R, dim), jnp.float32),
            pltpu.VMEM((_R, dim), jnp.float32),
            pltpu.VMEM((_R, dim), jnp.float32),
            pltpu.VMEM((_R, dim), jnp.float32),
            pltpu.SemaphoreType.DMA,
            pltpu.SemaphoreType.DMA,
            pltpu.SemaphoreType.DMA,
            pltpu.SemaphoreType.DMA,
        ],
    )(emb)
